# Initial kernel scaffold; baseline (speedup 1.0000x reference)
import jax
import jax.numpy as jnp
from jax.experimental import pallas as pl


def kernel(H, Q, z):
    raise NotImplementedError("write your pallas kernel here")


# speedup vs baseline: 8.1778x; 8.1778x over previous
import functools

import numpy as np
import jax
import jax.numpy as jnp
from jax.experimental import pallas as pl
from jax.experimental.pallas import tpu as pltpu

WAVELEN = 6.37e-07
NITER = 3
LAM = 0.01
BETA = 2000
N = 4
M = 512
M2INV = float(2.0 ** -18)

_interpret = False


@functools.cache
def _dft_mats():
    j = np.arange(M)
    jk = np.outer(j, j) % M
    ang = (2.0 * np.pi / M) * jk
    C = np.cos(ang).astype(np.float32)
    S = np.sin(ang).astype(np.float32)
    return jnp.asarray(C), jnp.asarray(S)


def _mm(x, y):
    return jax.lax.dot(x, y, precision=jax.lax.Precision.HIGHEST,
                       preferred_element_type=jnp.float32)


def _fft_plane(ar, ai, C, S):
    tr = _mm(ar, C) + _mm(ai, S)
    ti = _mm(ai, C) - _mm(ar, S)
    yr = _mm(C, tr) + _mm(S, ti)
    yi = _mm(C, ti) - _mm(S, tr)
    return yr, yi


def _ifft_plane(ar, ai, C, S):
    tr = _mm(ar, C) - _mm(ai, S)
    ti = _mm(ai, C) + _mm(ar, S)
    yr = (_mm(C, tr) - _mm(S, ti)) * M2INV
    yi = (_mm(C, ti) + _mm(S, tr)) * M2INV
    return yr, yi


def _tc_body(q, a, tcr, tci):
    arg = a[0, 0] * q[...]
    tcr[...] = jnp.cos(arg)
    tci[...] = jnp.sin(arg)


def _p1_first_body(hr, hi, c, s, qmfr, qmfi, magr):
    yr, yi = _fft_plane(hr[0], hi[0], c[...], s[...])
    qmfr[0] = yr
    qmfi[0] = yi
    magr[0] = jnp.sqrt(yr * yr + yi * yi)


def _p1_main_body(xfr, xfi, bmr, bmi, hr, hi, tcr, tci, c, s,
                  wro, wio, qmfr, qmfi, bro, bio, magr):
    C = c[...]
    S = s[...]
    tr = tcr[...]
    ti = tci[...]
    xr = xfr[0]
    xi = xfi[0]
    pr = xr * tr - xi * ti
    pi = xr * ti + xi * tr
    ur = pr + bmr[0]
    ui = pi + bmi[0]
    wr, wi = _ifft_plane(ur, ui, C, S)
    m = jnp.sqrt(wr * wr + wi * wi) + jnp.float32(1e-30)
    wr = wr / m
    wi = wi / m
    wro[0] = wr
    wio[0] = wi
    qr = hr[0] * wr - hi[0] * wi
    qi = hr[0] * wi + hi[0] * wr
    fr, fi = _fft_plane(qr, qi, C, S)
    qmfr[0] = fr
    qmfi[0] = fi
    br = fr - pr
    bi = fi - pi
    bro[0] = br
    bio[0] = bi
    magr[0] = jnp.sqrt(br * br + bi * bi)


def _p2_body(mag, th):
    f32 = jnp.float32

    def sbody(i, carry):
        t, pw = carry
        t2 = t + pw
        cand = jax.lax.bitcast_convert_type(t2, f32)
        cnt = jnp.int32(0)
        for n in range(N):
            cnt = cnt + jnp.sum((mag[n] >= cand).astype(jnp.int32))
        t = jax.lax.select(cnt >= BETA, t2, t)
        return (t, jax.lax.div(pw, jnp.int32(2)))

    t, _ = jax.lax.fori_loop(0, 31, sbody, (jnp.int32(0), jnp.int32(2 ** 30)))
    th[0, 0] = jax.lax.bitcast_convert_type(t, f32)


def _p3_body(br, bi, qmfr, qmfi, magr, th, tcr, tci, c, s,
             bmro, bmio, xfro, xfio):
    C = c[...]
    S = s[...]
    t = th[0, 0]
    keep = magr[0] >= t
    z = jnp.float32(0)
    bmr = jnp.where(keep, br[0], z)
    bmi = jnp.where(keep, bi[0], z)
    bmro[0] = bmr
    bmio[0] = bmi
    dr = qmfr[0] - bmr
    di = qmfi[0] - bmi
    tr = tcr[...]
    ti = tci[...]
    vr = dr * tr + di * ti
    vi = di * tr - dr * ti
    ir_, ii_ = _ifft_plane(vr, vi, C, S)
    keep2 = jnp.sqrt(ir_ * ir_ + ii_ * ii_) > jnp.float32(LAM)
    xr = jnp.where(keep2, ir_, z)
    xi = jnp.where(keep2, ii_, z)
    fr, fi = _fft_plane(xr, xi, C, S)
    xfro[0] = fr
    xfio[0] = fi


def _p3_mask_body(br, bi, magr, th, bmro, bmio):
    keep = magr[0] >= th[0, 0]
    z = jnp.float32(0)
    bmro[0] = jnp.where(keep, br[0], z)
    bmio[0] = jnp.where(keep, bi[0], z)


def _plane_spec():
    return pl.BlockSpec((1, M, M), lambda n: (n, 0, 0))


def _full_spec():
    return pl.BlockSpec((M, M), lambda n: (0, 0))


def _th_spec():
    return pl.BlockSpec(memory_space=pltpu.SMEM)


def _f32s(shape):
    return jax.ShapeDtypeStruct(shape, jnp.float32)


def _compute_tc(Q, a):
    return pl.pallas_call(
        _tc_body,
        out_shape=[_f32s((M, M)), _f32s((M, M))],
        interpret=_interpret,
    )(Q, a)


def _p1_first(hr, hi, C, S):
    return pl.pallas_call(
        _p1_first_body,
        grid=(N,),
        in_specs=[_plane_spec(), _plane_spec(), _full_spec(), _full_spec()],
        out_specs=[_plane_spec(), _plane_spec(), _plane_spec()],
        out_shape=[_f32s((N, M, M))] * 3,
        interpret=_interpret,
    )(hr, hi, C, S)


def _p1_main(xfr, xfi, bmr, bmi, hr, hi, tcr, tci, C, S):
    return pl.pallas_call(
        _p1_main_body,
        grid=(N,),
        in_specs=[_plane_spec()] * 6 + [_full_spec()] * 4,
        out_specs=[_plane_spec()] * 7,
        out_shape=[_f32s((N, M, M))] * 7,
        interpret=_interpret,
    )(xfr, xfi, bmr, bmi, hr, hi, tcr, tci, C, S)


def _p2(mag):
    return pl.pallas_call(
        _p2_body,
        out_shape=_f32s((1, 1)),
        out_specs=pl.BlockSpec(memory_space=pltpu.SMEM),
        interpret=_interpret,
    )(mag)


def _p3(br, bi, qmfr, qmfi, mag, th, tcr, tci, C, S):
    return pl.pallas_call(
        _p3_body,
        grid=(N,),
        in_specs=[_plane_spec()] * 5 + [_th_spec()] + [_full_spec()] * 4,
        out_specs=[_plane_spec()] * 4,
        out_shape=[_f32s((N, M, M))] * 4,
        interpret=_interpret,
    )(br, bi, qmfr, qmfi, mag, th, tcr, tci, C, S)


def _p3_mask(br, bi, mag, th):
    return pl.pallas_call(
        _p3_mask_body,
        grid=(N,),
        in_specs=[_plane_spec()] * 3 + [_th_spec()],
        out_specs=[_plane_spec()] * 2,
        out_shape=[_f32s((N, M, M))] * 2,
        interpret=_interpret,
    )(br, bi, mag, th)


def kernel(H, Q, z):
    Ht = jnp.transpose(H, (3, 0, 1, 2))
    hr, hi = Ht[0], Ht[1]
    a = (z[0] / np.float32(WAVELEN)).reshape(1, 1).astype(jnp.float32)
    C, S = _dft_mats()

    tcr, tci = _compute_tc(Q, a)

    qmfr, qmfi, mag = _p1_first(hr, hi, C, S)
    th = _p2(mag)
    bmr, bmi, xfr, xfi = _p3(qmfr, qmfi, qmfr, qmfi, mag, th, tcr, tci, C, S)

    _, _, qmfr, qmfi, br, bi, mag = _p1_main(
        xfr, xfi, bmr, bmi, hr, hi, tcr, tci, C, S)
    th = _p2(mag)
    bmr, bmi, xfr, xfi = _p3(br, bi, qmfr, qmfi, mag, th, tcr, tci, C, S)

    wr, wi, _, _, br, bi, mag = _p1_main(
        xfr, xfi, bmr, bmi, hr, hi, tcr, tci, C, S)
    th = _p2(mag)
    bmr, bmi = _p3_mask(br, bi, mag, th)

    W = jnp.stack([wr, wi], axis=-1)
    B = jnp.stack([bmr, bmi], axis=-1)
    return (W, B)


# speedup vs baseline: 10.0861x; 1.2334x over previous
import functools

import numpy as np
import jax
import jax.numpy as jnp
from jax.experimental import pallas as pl
from jax.experimental.pallas import tpu as pltpu

WAVELEN = 6.37e-07
NITER = 3
LAM = 0.01
BETA = 2000
N = 4
M = 512
M2INV = float(2.0 ** -18)

_interpret = False


@functools.cache
def _dft_mats():
    j = np.arange(M)
    jk = np.outer(j, j) % M
    ang = (2.0 * np.pi / M) * jk
    C = np.cos(ang)
    S = np.sin(ang)
    return (jnp.asarray(C.astype(np.float32)),
            jnp.asarray(S.astype(np.float32)),
            jnp.asarray((C - S).astype(np.float32)),
            jnp.asarray((C + S).astype(np.float32)))


def _mm(x, y):
    return jax.lax.dot(x, y, precision=jax.lax.Precision.HIGHEST,
                       preferred_element_type=jnp.float32)


def _fft_plane(ar, ai, C, S, CmS):
    m1 = _mm(ar, C)
    m2 = _mm(ai, S)
    m3 = _mm(ar + ai, CmS)
    tr = m1 + m2
    ti = m3 - m1 + m2
    p1 = _mm(C, tr)
    p2 = _mm(S, ti)
    p3 = _mm(CmS, tr + ti)
    return p1 + p2, p3 - p1 + p2


def _ifft_plane(ar, ai, C, S, CpS):
    m1 = _mm(ar, C)
    m2 = _mm(ai, S)
    m3 = _mm(ar + ai, CpS)
    tr = m1 - m2
    ti = m3 - m1 - m2
    p1 = _mm(C, tr)
    p2 = _mm(S, ti)
    p3 = _mm(CpS, tr + ti)
    return (p1 - p2) * M2INV, (p3 - p1 - p2) * M2INV


def _search_thresh(magf):
    f32 = jnp.float32

    def sbody(i, carry):
        t, pw = carry
        t2 = t + pw
        cand = jax.lax.bitcast_convert_type(t2, f32)
        cnt = jnp.int32(0)
        for n in range(N):
            cnt = cnt + jnp.sum((magf[n] >= cand).astype(jnp.int32))
        t = jax.lax.select(cnt >= BETA, t2, t)
        return (t, jax.lax.div(pw, jnp.int32(2)))

    t, _ = jax.lax.fori_loop(0, 31, sbody, (jnp.int32(0), jnp.int32(2 ** 30)))
    return jax.lax.bitcast_convert_type(t, f32)


def _tc_body(q, a, tcr, tci):
    arg = a[0, 0] * q[...]
    tcr[...] = jnp.cos(arg)
    tci[...] = jnp.sin(arg)


def _p1_first_body(hr, hi, c, s, cms, qmfr, qmfi, magr):
    yr, yi = _fft_plane(hr[0], hi[0], c[...], s[...], cms[...])
    qmfr[0] = yr
    qmfi[0] = yi
    magr[0] = jnp.sqrt(yr * yr + yi * yi)


def _p1_main_body(xfr, xfi, bmr, bmi, hr, hi, tcr, tci, c, s, cms, cps,
                  wro, wio, qmfr, qmfi, bro, bio, magr):
    C = c[...]
    S = s[...]
    tr = tcr[...]
    ti = tci[...]
    xr = xfr[0]
    xi = xfi[0]
    pr = xr * tr - xi * ti
    pi = xr * ti + xi * tr
    ur = pr + bmr[0]
    ui = pi + bmi[0]
    wr, wi = _ifft_plane(ur, ui, C, S, cps[...])
    m = jnp.sqrt(wr * wr + wi * wi) + jnp.float32(1e-30)
    wr = wr / m
    wi = wi / m
    wro[0] = wr
    wio[0] = wi
    qr = hr[0] * wr - hi[0] * wi
    qi = hr[0] * wi + hi[0] * wr
    fr, fi = _fft_plane(qr, qi, C, S, cms[...])
    qmfr[0] = fr
    qmfi[0] = fi
    br = fr - pr
    bi = fi - pi
    bro[0] = br
    bio[0] = bi
    magr[0] = jnp.sqrt(br * br + bi * bi)


def _p3_body(magf, br, bi, qmfr, qmfi, tcr, tci, c, s, cms, cps,
             bmro, bmio, xfro, xfio, th):
    n = pl.program_id(0)

    @pl.when(n == 0)
    def _():
        th[0] = _search_thresh(magf)

    C = c[...]
    S = s[...]
    t = th[0]
    brv = br[0]
    biv = bi[0]
    keep = jnp.sqrt(brv * brv + biv * biv) >= t
    z = jnp.float32(0)
    bmr = jnp.where(keep, brv, z)
    bmi = jnp.where(keep, biv, z)
    bmro[0] = bmr
    bmio[0] = bmi
    dr = qmfr[0] - bmr
    di = qmfi[0] - bmi
    tr = tcr[...]
    ti = tci[...]
    vr = dr * tr + di * ti
    vi = di * tr - dr * ti
    ir_, ii_ = _ifft_plane(vr, vi, C, S, cps[...])
    keep2 = jnp.sqrt(ir_ * ir_ + ii_ * ii_) > jnp.float32(LAM)
    xr = jnp.where(keep2, ir_, z)
    xi = jnp.where(keep2, ii_, z)
    fr, fi = _fft_plane(xr, xi, C, S, cms[...])
    xfro[0] = fr
    xfio[0] = fi


def _p3_mask_body(magf, br, bi, bmro, bmio, th):
    n = pl.program_id(0)

    @pl.when(n == 0)
    def _():
        th[0] = _search_thresh(magf)

    t = th[0]
    brv = br[0]
    biv = bi[0]
    keep = jnp.sqrt(brv * brv + biv * biv) >= t
    z = jnp.float32(0)
    bmro[0] = jnp.where(keep, brv, z)
    bmio[0] = jnp.where(keep, biv, z)


def _plane_spec():
    return pl.BlockSpec((1, M, M), lambda n: (n, 0, 0))


def _full_spec():
    return pl.BlockSpec((M, M), lambda n: (0, 0))


def _mag_spec():
    return pl.BlockSpec((N, M, M), lambda n: (0, 0, 0))


def _f32s(shape):
    return jax.ShapeDtypeStruct(shape, jnp.float32)


def _compute_tc(Q, a):
    return pl.pallas_call(
        _tc_body,
        out_shape=[_f32s((M, M)), _f32s((M, M))],
        interpret=_interpret,
    )(Q, a)


def _p1_first(hr, hi, C, S, CmS):
    return pl.pallas_call(
        _p1_first_body,
        grid=(N,),
        in_specs=[_plane_spec()] * 2 + [_full_spec()] * 3,
        out_specs=[_plane_spec()] * 3,
        out_shape=[_f32s((N, M, M))] * 3,
        interpret=_interpret,
    )(hr, hi, C, S, CmS)


def _p1_main(xfr, xfi, bmr, bmi, hr, hi, tcr, tci, C, S, CmS, CpS):
    return pl.pallas_call(
        _p1_main_body,
        grid=(N,),
        in_specs=[_plane_spec()] * 6 + [_full_spec()] * 6,
        out_specs=[_plane_spec()] * 7,
        out_shape=[_f32s((N, M, M))] * 7,
        interpret=_interpret,
    )(xfr, xfi, bmr, bmi, hr, hi, tcr, tci, C, S, CmS, CpS)


def _p3(mag, br, bi, qmfr, qmfi, tcr, tci, C, S, CmS, CpS):
    return pl.pallas_call(
        _p3_body,
        grid=(N,),
        in_specs=[_mag_spec()] + [_plane_spec()] * 4 + [_full_spec()] * 6,
        out_specs=[_plane_spec()] * 4,
        out_shape=[_f32s((N, M, M))] * 4,
        scratch_shapes=[pltpu.SMEM((1,), jnp.float32)],
        interpret=_interpret,
    )(mag, br, bi, qmfr, qmfi, tcr, tci, C, S, CmS, CpS)


def _p3_mask(mag, br, bi):
    return pl.pallas_call(
        _p3_mask_body,
        grid=(N,),
        in_specs=[_mag_spec()] + [_plane_spec()] * 2,
        out_specs=[_plane_spec()] * 2,
        out_shape=[_f32s((N, M, M))] * 2,
        scratch_shapes=[pltpu.SMEM((1,), jnp.float32)],
        interpret=_interpret,
    )(mag, br, bi)


def kernel(H, Q, z):
    Ht = jnp.transpose(H, (3, 0, 1, 2))
    hr, hi = Ht[0], Ht[1]
    a = (z[0] / np.float32(WAVELEN)).reshape(1, 1).astype(jnp.float32)
    C, S, CmS, CpS = _dft_mats()

    tcr, tci = _compute_tc(Q, a)

    qmfr, qmfi, mag = _p1_first(hr, hi, C, S, CmS)
    bmr, bmi, xfr, xfi = _p3(mag, qmfr, qmfi, qmfr, qmfi, tcr, tci, C, S, CmS, CpS)

    _, _, qmfr, qmfi, br, bi, mag = _p1_main(
        xfr, xfi, bmr, bmi, hr, hi, tcr, tci, C, S, CmS, CpS)
    bmr, bmi, xfr, xfi = _p3(mag, br, bi, qmfr, qmfi, tcr, tci, C, S, CmS, CpS)

    wr, wi, _, _, br, bi, mag = _p1_main(
        xfr, xfi, bmr, bmi, hr, hi, tcr, tci, C, S, CmS, CpS)
    bmr, bmi = _p3_mask(mag, br, bi)

    W = jnp.stack([wr, wi], axis=-1)
    B = jnp.stack([bmr, bmi], axis=-1)
    return (W, B)
